# baseline (speedup 1.0000x reference)
import jax
import jax.numpy as jnp
from jax import lax
from jax.experimental import pallas as pl
from jax.experimental.pallas import tpu as pltpu
from jax.experimental.pallas import tpu_sc as plsc

BATCH = 16384
EMBED_DIM = 32
NUM_WORKERS = 32
B_PER_W = BATCH // NUM_WORKERS
LANES = 16

PAD_MINOR = 1000064
FLAT = EMBED_DIM * PAD_MINOR
M_PER_W = 244 * 128
TAIL_OFF = NUM_WORKERS * M_PER_W
TAIL = 1000000 - TAIL_OFF


def _detile_body(utab3, itab3, uflat, iflat,
                 ub0, ib0, ub1, ib1, tbuf,
                 ru0, ri0, ru1, ri1, wu0, wi0, wu1, wi1):
    wid = lax.axis_index("s") * 2 + lax.axis_index("c")
    m0 = wid * M_PER_W
    dummy = utab3.at[0, 0, pl.ds(0, M_PER_W)]
    bufs = ((ub0, ib0, ru0, ri0, wu0, wi0), (ub1, ib1, ru1, ri1, wu1, wi1))

    def k_body(k, carry):
        for half, (ub, ib, ru, ri, wu, wi) in enumerate(bufs):
            c = 2 * k + half

            @pl.when(k >= 1)
            def _():
                pltpu.make_async_copy(dummy, ub, wu).wait()
                pltpu.make_async_copy(dummy, ib, wi).wait()
            pltpu.async_copy(utab3.at[0, c, pl.ds(m0, M_PER_W)], ub, ru)
            pltpu.async_copy(itab3.at[0, c, pl.ds(m0, M_PER_W)], ib, ri)
        for half, (ub, ib, ru, ri, wu, wi) in enumerate(bufs):
            c = 2 * k + half
            pltpu.make_async_copy(dummy, ub, ru).wait()
            pltpu.async_copy(ub, uflat.at[pl.ds(c * PAD_MINOR + m0, M_PER_W)],
                             wu)
            pltpu.make_async_copy(dummy, ib, ri).wait()
            pltpu.async_copy(ib, iflat.at[pl.ds(c * PAD_MINOR + m0, M_PER_W)],
                             wi)
        return carry

    lax.fori_loop(0, EMBED_DIM // 2, k_body, 0)
    for ub, ib, ru, ri, wu, wi in bufs:
        pltpu.make_async_copy(dummy, ub, wu).wait()
        pltpu.make_async_copy(dummy, ib, wi).wait()

    @pl.when(wid == 0)
    def _():
        def t_body(c, carry):
            pltpu.sync_copy(utab3.at[0, c, pl.ds(TAIL_OFF, TAIL)], tbuf)
            pltpu.sync_copy(tbuf, uflat.at[pl.ds(c * PAD_MINOR + TAIL_OFF, TAIL)])
            pltpu.sync_copy(itab3.at[0, c, pl.ds(TAIL_OFF, TAIL)], tbuf)
            pltpu.sync_copy(tbuf, iflat.at[pl.ds(c * PAD_MINOR + TAIL_OFF, TAIL)])
            return carry
        lax.fori_loop(0, EMBED_DIM, t_body, 0)


def _score_body(uids_hbm, iids_hbm, uflat, iflat, out_hbm,
                uidx_v, iidx_v, u_v, i_v, out_v, sem):
    wid = lax.axis_index("s") * 2 + lax.axis_index("c")
    base = wid * B_PER_W

    pltpu.sync_copy(uids_hbm.at[pl.ds(base, B_PER_W)], uidx_v)
    pltpu.sync_copy(iids_hbm.at[pl.ds(base, B_PER_W)], iidx_v)

    def gather_body(g, carry):
        sl = pl.ds(g * LANES, LANES)
        uvec = uidx_v[sl]
        ivec = iidx_v[sl]
        for c in range(EMBED_DIM):
            off = jnp.int32(c * PAD_MINOR)
            pltpu.async_copy(uflat.at[uvec + off], u_v.at[c, sl], sem)
            pltpu.async_copy(iflat.at[ivec + off], i_v.at[c, sl], sem)
        return carry

    lax.fori_loop(0, B_PER_W // LANES, gather_body, 0)

    def drain_body(c, carry):
        pltpu.make_async_copy(
            uflat.at[pl.ds(0, B_PER_W)], u_v.at[c, :], sem).wait()
        pltpu.make_async_copy(
            iflat.at[pl.ds(0, B_PER_W)], i_v.at[c, :], sem).wait()
        return carry

    lax.fori_loop(0, EMBED_DIM, drain_body, 0)

    iota16 = lax.iota(jnp.int32, LANES)

    def group_body(g, carry):
        sl = pl.ds(g * LANES, LANES)
        acc = jnp.zeros((LANES,), jnp.float32)
        for c in range(EMBED_DIM):
            acc = acc + u_v[c, sl] * i_v[c, sl]
        e = jnp.exp(-jnp.abs(acc))
        num = jnp.where(acc >= 0, jnp.ones_like(acc), e)
        plsc.store_scatter(out_v, [g * LANES + iota16], num / (1.0 + e))
        return carry

    lax.fori_loop(0, B_PER_W // LANES, group_body, 0)

    pltpu.sync_copy(out_v, out_hbm.at[pl.ds(base, B_PER_W)])


@jax.jit
def kernel(user_ids, item_ids, user_table, item_table):
    uids = user_ids.astype(jnp.int32)
    iids = item_ids.astype(jnp.int32)
    utab3 = user_table.T[None]
    itab3 = item_table.T[None]

    mesh = plsc.VectorSubcoreMesh(core_axis_name="c", subcore_axis_name="s")

    detile = pl.kernel(
        _detile_body, mesh=mesh,
        out_type=(jax.ShapeDtypeStruct((FLAT,), jnp.float32),
                  jax.ShapeDtypeStruct((FLAT,), jnp.float32)),
        compiler_params=pltpu.CompilerParams(needs_layout_passes=False),
        scratch_types=[
            pltpu.VMEM((M_PER_W,), jnp.float32),
            pltpu.VMEM((M_PER_W,), jnp.float32),
            pltpu.VMEM((M_PER_W,), jnp.float32),
            pltpu.VMEM((M_PER_W,), jnp.float32),
            pltpu.VMEM((TAIL,), jnp.float32),
        ] + [pltpu.SemaphoreType.DMA] * 8,
    )
    uflat, iflat = detile(utab3, itab3)

    score = pl.kernel(
        _score_body, mesh=mesh,
        out_type=jax.ShapeDtypeStruct((BATCH,), jnp.float32),
        compiler_params=pltpu.CompilerParams(
            use_tc_tiling_on_sc=False, needs_layout_passes=False),
        scratch_types=[
            pltpu.VMEM((B_PER_W,), jnp.int32),
            pltpu.VMEM((B_PER_W,), jnp.int32),
            pltpu.VMEM((EMBED_DIM, B_PER_W), jnp.float32),
            pltpu.VMEM((EMBED_DIM, B_PER_W), jnp.float32),
            pltpu.VMEM((B_PER_W,), jnp.float32),
            pltpu.SemaphoreType.DMA,
        ],
    )
    return score(uids, iids, uflat, iflat)
